# baseline (speedup 1.0000x reference)
import functools

import jax
import jax.numpy as jnp
from jax import lax
from jax.experimental import pallas as pl
from jax.experimental.pallas import tpu as pltpu, tpu_sc as plsc

_NC = 2
_NS = 16
_L = 16
_C = 1536


def _pack2(a, b):
    ua = lax.bitcast_convert_type(
        a.astype(jnp.bfloat16).astype(jnp.float32), jnp.uint32)
    ub = lax.bitcast_convert_type(
        b.astype(jnp.bfloat16).astype(jnp.float32), jnp.uint32)
    return lax.bitcast_convert_type(ua | (ub >> 16), jnp.int32)


def _table_body(w_ref, v_ref, th_ref, o_w01, o_w2t0, o_t12, o_sc):
    w0, w1, w2 = w_ref[0], w_ref[1], w_ref[2]
    v0, v1, v2 = v_ref[0], v_ref[1], v_ref[2]
    th = th_ref[...]
    s = jnp.sin(th)
    c1 = 1.0 - jnp.cos(th)
    tms = th - s
    a0 = w1 * v2 - w2 * v1
    a1 = w2 * v0 - w0 * v2
    a2 = w0 * v1 - w1 * v0
    b0 = w1 * a2 - w2 * a1
    b1 = w2 * a0 - w0 * a2
    b2 = w0 * a1 - w1 * a0
    t0 = th * v0 + c1 * a0 + tms * b0
    t1 = th * v1 + c1 * a1 + tms * b1
    t2 = th * v2 + c1 * a2 + tms * b2
    o_w01[...] = _pack2(w0, w1)
    o_w2t0[...] = _pack2(w2, t0)
    o_t12[...] = _pack2(t1, t2)
    o_sc[...] = _pack2(s, c1)


def _build_tables(w, v, theta):
    n = theta.shape[0]
    r = n // 128
    wt = w.T.reshape(3, r, 128)
    vt = v.T.reshape(3, r, 128)
    th = theta.reshape(r, 128)
    o = jax.ShapeDtypeStruct((r, 128), jnp.int32)
    t01, t2t0, t12, tsc = pl.pallas_call(
        _table_body, out_shape=[o, o, o, o])(wt, vt, th)
    return t01.reshape(n), t2t0.reshape(n), t12.reshape(n), tsc.reshape(n)


def _make_sc_apply(B, n):
    W = _NC * _NS
    K = -(-B // (_C * W))
    last = (B // 128) * 128 - _C + 128
    idx_mask = n - 1

    mesh = plsc.VectorSubcoreMesh(
        core_axis_name="c", subcore_axis_name="s",
        num_cores=_NC, num_subcores=_NS)

    @functools.partial(
        pl.kernel,
        out_type=jax.ShapeDtypeStruct((8, B), jnp.float32),
        mesh=mesh,
        scratch_types=[
            pltpu.VMEM((n,), jnp.int32),
            pltpu.VMEM((n,), jnp.int32),
            pltpu.VMEM((n,), jnp.int32),
            pltpu.VMEM((n,), jnp.int32),
            [pltpu.VMEM((6, _C), jnp.float32) for _ in range(2)],
            [pltpu.VMEM((6, _C), jnp.float32) for _ in range(2)],
            [pltpu.VMEM((_C,), jnp.int32) for _ in range(2)],
            [pltpu.VMEM((_C,), jnp.float32) for _ in range(2)],
            [pltpu.SemaphoreType.DMA for _ in range(2)],
            [pltpu.SemaphoreType.DMA for _ in range(2)],
        ],
        compiler_params=pltpu.CompilerParams(
            needs_layout_passes=False, use_tc_tiling_on_sc=True),
    )
    def sc_apply(t01_hbm, t2t0_hbm, t12_hbm, tsc_hbm, raysT_hbm, idx_hbm,
                 mask_hbm, out_hbm, t01_v, t2t0_v, t12_v, tsc_v,
                 rin, rout, idxv, mskv, sin_, sout):
        wid = lax.axis_index("s") * _NC + lax.axis_index("c")
        pltpu.sync_copy(t01_hbm, t01_v)
        pltpu.sync_copy(t2t0_hbm, t2t0_v)
        pltpu.sync_copy(t12_hbm, t12_v)
        pltpu.sync_copy(tsc_hbm, tsc_v)

        def hi(p):
            return plsc.bitcast(p & jnp.int32(-65536), jnp.float32)

        def lo(p):
            return plsc.bitcast(p << 16, jnp.float32)

        def chunk_start(c):
            return jnp.minimum((wid * K + jnp.minimum(c, K - 1)) * _C, last)

        def in_start(c, b):
            start = chunk_start(c)
            pltpu.async_copy(idx_hbm.at[pl.ds(start, _C)], idxv[b], sin_[b])
            pltpu.async_copy(mask_hbm.at[pl.ds(start, _C)], mskv[b], sin_[b])
            pltpu.async_copy(
                raysT_hbm.at[pl.ds(0, 6), pl.ds(start, _C)], rin[b], sin_[b])

        def in_wait(c, b):
            start = chunk_start(c)
            pltpu.make_async_copy(
                idx_hbm.at[pl.ds(start, _C)], idxv[b], sin_[b]).wait()
            pltpu.make_async_copy(
                mask_hbm.at[pl.ds(start, _C)], mskv[b], sin_[b]).wait()
            pltpu.make_async_copy(
                raysT_hbm.at[pl.ds(0, 6), pl.ds(start, _C)], rin[b],
                sin_[b]).wait()

        def out_start(c, b):
            start = chunk_start(c)
            pltpu.async_copy(
                rout[b], out_hbm.at[pl.ds(0, 6), pl.ds(start, _C)], sout[b])
            pltpu.async_copy(
                raysT_hbm.at[pl.ds(6, 2), pl.ds(start, _C)],
                out_hbm.at[pl.ds(6, 2), pl.ds(start, _C)], sout[b])

        def out_wait(c, b):
            start = chunk_start(c)
            pltpu.make_async_copy(
                rout[b], out_hbm.at[pl.ds(0, 6), pl.ds(start, _C)],
                sout[b]).wait()
            pltpu.make_async_copy(
                raysT_hbm.at[pl.ds(6, 2), pl.ds(start, _C)],
                out_hbm.at[pl.ds(6, 2), pl.ds(start, _C)], sout[b]).wait()

        def compute(b):
            ray_i = rin[b]
            ray_o = rout[b]
            idx_v = idxv[b]
            mask_v = mskv[b]

            def do_group(gi):
                sl = pl.ds(gi * _L, _L)
                ix = idx_v[sl] & idx_mask
                p01 = plsc.load_gather(t01_v, [ix])
                p2t0 = plsc.load_gather(t2t0_v, [ix])
                pt12 = plsc.load_gather(t12_v, [ix])
                psc = plsc.load_gather(tsc_v, [ix])
                mf = mask_v[sl]
                w0 = hi(p01)
                w1 = lo(p01)
                w2 = hi(p2t0)
                t0 = lo(p2t0)
                t1 = hi(pt12)
                t2 = lo(pt12)
                sm = hi(psc) * mf
                c1m = lo(psc) * mf
                ox = ray_i[0, sl]
                oy = ray_i[1, sl]
                oz = ray_i[2, sl]
                dx = ray_i[3, sl]
                dy = ray_i[4, sl]
                dz = ray_i[5, sl]
                cx = w1 * dz - w2 * dy
                cy = w2 * dx - w0 * dz
                cz = w0 * dy - w1 * dx
                ex = w1 * cz - w2 * cy
                ey = w2 * cx - w0 * cz
                ez = w0 * cy - w1 * cx
                ray_o[0, sl] = ox + t0 * mf
                ray_o[1, sl] = oy + t1 * mf
                ray_o[2, sl] = oz + t2 * mf
                ray_o[3, sl] = dx + sm * cx + c1m * ex
                ray_o[4, sl] = dy + sm * cy + c1m * ey
                ray_o[5, sl] = dz + sm * cz + c1m * ez

            def group(g, c2):
                do_group(g * 2)
                do_group(g * 2 + 1)
                return c2

            lax.fori_loop(0, _C // (2 * _L), group, 0)

        in_start(0, 0)
        in_start(1, 1)
        in_wait(0, 0)
        compute(0)
        out_start(0, 0)
        in_start(2, 0)
        in_wait(1, 1)
        compute(1)
        out_start(1, 1)
        in_start(3, 1)

        def pair(j, carry):
            a = 2 * j
            in_wait(a, 0)
            out_wait(a - 2, 0)
            compute(0)
            out_start(a, 0)
            in_start(a + 2, 0)
            in_wait(a + 1, 1)
            out_wait(a - 1, 1)
            compute(1)
            out_start(a + 1, 1)
            in_start(a + 3, 1)
            return carry

        J = (K - 1) // 2
        lax.fori_loop(1, J + 1, pair, 0)
        in_wait(2 * J + 2, 0)
        in_wait(2 * J + 3, 1)
        out_wait(2 * J, 0)
        out_wait(2 * J + 1, 1)

    return sc_apply


def kernel(w, v, theta, rays, image_indices, depth_mask):
    B = rays.shape[0]
    n = theta.shape[0]
    t01, t2t0, t12, tsc = _build_tables(
        w.astype(jnp.float32), v.astype(jnp.float32),
        theta.astype(jnp.float32))
    raysT = rays.T
    idx = image_indices.reshape(-1).astype(jnp.int32)
    maskf = (depth_mask.reshape(-1) == 1).astype(jnp.float32)
    sc_apply = _make_sc_apply(B, n)
    outT = sc_apply(t01, t2t0, t12, tsc, raysT, idx, maskf)
    return outT.T


# speedup vs baseline: 2.3876x; 2.3876x over previous
import functools

import jax
import jax.numpy as jnp
from jax import lax
from jax.experimental import pallas as pl
from jax.experimental.pallas import tpu as pltpu, tpu_sc as plsc

_NC = 2
_NS = 16
_L = 16
_C = 1536


def _pack2(a, b):
    ua = lax.bitcast_convert_type(
        a.astype(jnp.bfloat16).astype(jnp.float32), jnp.uint32)
    ub = lax.bitcast_convert_type(
        b.astype(jnp.bfloat16).astype(jnp.float32), jnp.uint32)
    return lax.bitcast_convert_type(ua | (ub >> 16), jnp.int32)


def _table_body(w_ref, v_ref, th_ref, o_w01, o_w2t0, o_t12, o_sc):
    w0, w1, w2 = w_ref[0], w_ref[1], w_ref[2]
    v0, v1, v2 = v_ref[0], v_ref[1], v_ref[2]
    th = th_ref[...]
    s = jnp.sin(th)
    c1 = 1.0 - jnp.cos(th)
    tms = th - s
    a0 = w1 * v2 - w2 * v1
    a1 = w2 * v0 - w0 * v2
    a2 = w0 * v1 - w1 * v0
    b0 = w1 * a2 - w2 * a1
    b1 = w2 * a0 - w0 * a2
    b2 = w0 * a1 - w1 * a0
    t0 = th * v0 + c1 * a0 + tms * b0
    t1 = th * v1 + c1 * a1 + tms * b1
    t2 = th * v2 + c1 * a2 + tms * b2
    o_w01[...] = _pack2(w0, w1)
    o_w2t0[...] = _pack2(w2, t0)
    o_t12[...] = _pack2(t1, t2)
    o_sc[...] = _pack2(s, c1)


def _build_tables(w, v, theta):
    n = theta.shape[0]
    r = n // 128
    wt = w.T.reshape(3, r, 128)
    vt = v.T.reshape(3, r, 128)
    th = theta.reshape(r, 128)
    o = jax.ShapeDtypeStruct((r, 128), jnp.int32)
    t01, t2t0, t12, tsc = pl.pallas_call(
        _table_body, out_shape=[o, o, o, o])(wt, vt, th)
    return t01.reshape(n), t2t0.reshape(n), t12.reshape(n), tsc.reshape(n)


def _make_sc_apply(B, n):
    W = _NC * _NS
    K = -(-B // (_C * W))
    last = (B // 128) * 128 - _C + 128
    idx_mask = n - 1

    mesh = plsc.VectorSubcoreMesh(
        core_axis_name="c", subcore_axis_name="s",
        num_cores=_NC, num_subcores=_NS)

    @functools.partial(
        pl.kernel,
        out_type=jax.ShapeDtypeStruct((8, B), jnp.float32),
        mesh=mesh,
        scratch_types=[
            pltpu.VMEM((n,), jnp.int32),
            pltpu.VMEM((n,), jnp.int32),
            pltpu.VMEM((n,), jnp.int32),
            pltpu.VMEM((n,), jnp.int32),
            [pltpu.VMEM((8, _C), jnp.float32) for _ in range(2)],
            [pltpu.VMEM((8, _C), jnp.float32) for _ in range(2)],
            [pltpu.VMEM((_C,), jnp.int32) for _ in range(2)],
            [pltpu.VMEM((_C,), jnp.float32) for _ in range(2)],
            [pltpu.SemaphoreType.DMA for _ in range(2)],
            [pltpu.SemaphoreType.DMA for _ in range(2)],
        ],
        compiler_params=pltpu.CompilerParams(
            needs_layout_passes=False, use_tc_tiling_on_sc=True),
    )
    def sc_apply(t01_hbm, t2t0_hbm, t12_hbm, tsc_hbm, raysT_hbm, idx_hbm,
                 mask_hbm, out_hbm, t01_v, t2t0_v, t12_v, tsc_v,
                 rin, rout, idxv, mskv, sin_, sout):
        wid = lax.axis_index("s") * _NC + lax.axis_index("c")
        pltpu.sync_copy(t01_hbm, t01_v)
        pltpu.sync_copy(t2t0_hbm, t2t0_v)
        pltpu.sync_copy(t12_hbm, t12_v)
        pltpu.sync_copy(tsc_hbm, tsc_v)

        def hi(p):
            return plsc.bitcast(p & jnp.int32(-65536), jnp.float32)

        def lo(p):
            return plsc.bitcast(p << 16, jnp.float32)

        def chunk_start(c):
            return jnp.minimum((wid * K + jnp.minimum(c, K - 1)) * _C, last)

        def in_start(c, b):
            start = chunk_start(c)
            pltpu.async_copy(idx_hbm.at[pl.ds(start, _C)], idxv[b], sin_[b])
            pltpu.async_copy(mask_hbm.at[pl.ds(start, _C)], mskv[b], sin_[b])
            pltpu.async_copy(raysT_hbm.at[:, pl.ds(start, _C)], rin[b],
                             sin_[b])

        def in_wait(c, b):
            start = chunk_start(c)
            pltpu.make_async_copy(
                idx_hbm.at[pl.ds(start, _C)], idxv[b], sin_[b]).wait()
            pltpu.make_async_copy(
                mask_hbm.at[pl.ds(start, _C)], mskv[b], sin_[b]).wait()
            pltpu.make_async_copy(
                raysT_hbm.at[:, pl.ds(start, _C)], rin[b], sin_[b]).wait()

        def out_start(c, b):
            start = chunk_start(c)
            pltpu.async_copy(rout[b], out_hbm.at[:, pl.ds(start, _C)],
                             sout[b])

        def out_wait(c, b):
            start = chunk_start(c)
            pltpu.make_async_copy(
                rout[b], out_hbm.at[:, pl.ds(start, _C)], sout[b]).wait()

        def compute(b):
            ray_i = rin[b]
            ray_o = rout[b]
            idx_v = idxv[b]
            mask_v = mskv[b]

            def do_group(gi):
                sl = pl.ds(gi * _L, _L)
                ix = idx_v[sl] & idx_mask
                p01 = plsc.load_gather(t01_v, [ix])
                p2t0 = plsc.load_gather(t2t0_v, [ix])
                pt12 = plsc.load_gather(t12_v, [ix])
                psc = plsc.load_gather(tsc_v, [ix])
                mf = mask_v[sl]
                w0 = hi(p01)
                w1 = lo(p01)
                w2 = hi(p2t0)
                t0 = lo(p2t0)
                t1 = hi(pt12)
                t2 = lo(pt12)
                sm = hi(psc) * mf
                c1m = lo(psc) * mf
                ox = ray_i[0, sl]
                oy = ray_i[1, sl]
                oz = ray_i[2, sl]
                dx = ray_i[3, sl]
                dy = ray_i[4, sl]
                dz = ray_i[5, sl]
                cx = w1 * dz - w2 * dy
                cy = w2 * dx - w0 * dz
                cz = w0 * dy - w1 * dx
                ex = w1 * cz - w2 * cy
                ey = w2 * cx - w0 * cz
                ez = w0 * cy - w1 * cx
                ray_o[0, sl] = ox + t0 * mf
                ray_o[1, sl] = oy + t1 * mf
                ray_o[2, sl] = oz + t2 * mf
                ray_o[3, sl] = dx + sm * cx + c1m * ex
                ray_o[4, sl] = dy + sm * cy + c1m * ey
                ray_o[5, sl] = dz + sm * cz + c1m * ez
                ray_o[6, sl] = ray_i[6, sl]
                ray_o[7, sl] = ray_i[7, sl]

            def group(g, c2):
                do_group(g * 2)
                do_group(g * 2 + 1)
                return c2

            lax.fori_loop(0, _C // (2 * _L), group, 0)

        in_start(0, 0)
        in_start(1, 1)
        in_wait(0, 0)
        compute(0)
        out_start(0, 0)
        in_start(2, 0)
        in_wait(1, 1)
        compute(1)
        out_start(1, 1)
        in_start(3, 1)

        def pair(j, carry):
            a = 2 * j
            in_wait(a, 0)
            out_wait(a - 2, 0)
            compute(0)
            out_start(a, 0)
            in_start(a + 2, 0)
            in_wait(a + 1, 1)
            out_wait(a - 1, 1)
            compute(1)
            out_start(a + 1, 1)
            in_start(a + 3, 1)
            return carry

        J = (K - 1) // 2
        lax.fori_loop(1, J + 1, pair, 0)
        in_wait(2 * J + 2, 0)
        in_wait(2 * J + 3, 1)
        out_wait(2 * J, 0)
        out_wait(2 * J + 1, 1)

    return sc_apply


def kernel(w, v, theta, rays, image_indices, depth_mask):
    B = rays.shape[0]
    n = theta.shape[0]
    t01, t2t0, t12, tsc = _build_tables(
        w.astype(jnp.float32), v.astype(jnp.float32),
        theta.astype(jnp.float32))
    raysT = rays.T
    idx = image_indices.reshape(-1).astype(jnp.int32)
    maskf = (depth_mask.reshape(-1) == 1).astype(jnp.float32)
    sc_apply = _make_sc_apply(B, n)
    outT = sc_apply(t01, t2t0, t12, tsc, raysT, idx, maskf)
    return outT.T


# speedup vs baseline: 2.9892x; 1.2520x over previous
import functools

import jax
import jax.numpy as jnp
from jax import lax
from jax.experimental import pallas as pl
from jax.experimental.pallas import tpu as pltpu, tpu_sc as plsc

_NC = 2
_NS = 16
_L = 16
_C = 1536


def _pack2(a, b):
    ua = lax.bitcast_convert_type(
        a.astype(jnp.bfloat16).astype(jnp.float32), jnp.uint32)
    ub = lax.bitcast_convert_type(
        b.astype(jnp.bfloat16).astype(jnp.float32), jnp.uint32)
    return lax.bitcast_convert_type(ua | (ub >> 16), jnp.int32)


def _table_body(w_ref, v_ref, th_ref, o_w01, o_w2t0, o_t12, o_sc):
    w0, w1, w2 = w_ref[0], w_ref[1], w_ref[2]
    v0, v1, v2 = v_ref[0], v_ref[1], v_ref[2]
    th = th_ref[...]
    s = jnp.sin(th)
    c1 = 1.0 - jnp.cos(th)
    tms = th - s
    a0 = w1 * v2 - w2 * v1
    a1 = w2 * v0 - w0 * v2
    a2 = w0 * v1 - w1 * v0
    b0 = w1 * a2 - w2 * a1
    b1 = w2 * a0 - w0 * a2
    b2 = w0 * a1 - w1 * a0
    t0 = th * v0 + c1 * a0 + tms * b0
    t1 = th * v1 + c1 * a1 + tms * b1
    t2 = th * v2 + c1 * a2 + tms * b2
    o_w01[...] = _pack2(w0, w1)
    o_w2t0[...] = _pack2(w2, t0)
    o_t12[...] = _pack2(t1, t2)
    o_sc[...] = _pack2(s, c1)


def _build_tables(w, v, theta):
    n = theta.shape[0]
    r = n // 128
    wt = w.T.reshape(3, r, 128)
    vt = v.T.reshape(3, r, 128)
    th = theta.reshape(r, 128)
    o = jax.ShapeDtypeStruct((r, 128), jnp.int32)
    t01, t2t0, t12, tsc = pl.pallas_call(
        _table_body, out_shape=[o, o, o, o])(wt, vt, th)
    return t01.reshape(n), t2t0.reshape(n), t12.reshape(n), tsc.reshape(n)


def _make_sc_apply(B, n):
    W = _NC * _NS
    K = -(-B // (_C * W))
    last = (B // 128) * 128 - _C + 128
    idx_mask = n - 1

    mesh = plsc.VectorSubcoreMesh(
        core_axis_name="c", subcore_axis_name="s",
        num_cores=_NC, num_subcores=_NS)

    @functools.partial(
        pl.kernel,
        out_type=jax.ShapeDtypeStruct((8, B), jnp.float32),
        mesh=mesh,
        scratch_types=[
            pltpu.VMEM((n,), jnp.int32),
            pltpu.VMEM((n,), jnp.int32),
            pltpu.VMEM((n,), jnp.int32),
            pltpu.VMEM((n,), jnp.int32),
            [pltpu.VMEM((8, _C), jnp.float32) for _ in range(2)],
            [pltpu.VMEM((8, _C), jnp.float32) for _ in range(2)],
            [pltpu.VMEM((_C,), jnp.int32) for _ in range(2)],
            [pltpu.VMEM((_C,), jnp.float32) for _ in range(2)],
            [pltpu.SemaphoreType.DMA for _ in range(2)],
            [pltpu.SemaphoreType.DMA for _ in range(2)],
        ],
        compiler_params=pltpu.CompilerParams(
            needs_layout_passes=False, use_tc_tiling_on_sc=True),
    )
    def sc_apply(t01_hbm, t2t0_hbm, t12_hbm, tsc_hbm, raysT_hbm, idx_hbm,
                 mask_hbm, out_hbm, t01_v, t2t0_v, t12_v, tsc_v,
                 rin, rout, idxv, mskv, sin_, sout):
        wid = lax.axis_index("s") * _NC + lax.axis_index("c")
        pltpu.sync_copy(t01_hbm, t01_v)
        pltpu.sync_copy(t2t0_hbm, t2t0_v)
        pltpu.sync_copy(t12_hbm, t12_v)
        pltpu.sync_copy(tsc_hbm, tsc_v)

        def hi(p):
            return plsc.bitcast(p & jnp.int32(-65536), jnp.float32)

        def lo(p):
            return plsc.bitcast(p << 16, jnp.float32)

        def chunk_start(c):
            return jnp.minimum((wid * K + jnp.minimum(c, K - 1)) * _C, last)

        def in_start(c, b):
            start = chunk_start(c)
            pltpu.async_copy(idx_hbm.at[pl.ds(start, _C)], idxv[b], sin_[b])
            pltpu.async_copy(mask_hbm.at[pl.ds(start, _C)], mskv[b], sin_[b])
            pltpu.async_copy(raysT_hbm.at[:, pl.ds(start, _C)], rin[b],
                             sin_[b])

        def in_wait(c, b):
            start = chunk_start(c)
            pltpu.make_async_copy(
                idx_hbm.at[pl.ds(start, _C)], idxv[b], sin_[b]).wait()
            pltpu.make_async_copy(
                mask_hbm.at[pl.ds(start, _C)], mskv[b], sin_[b]).wait()
            pltpu.make_async_copy(
                raysT_hbm.at[:, pl.ds(start, _C)], rin[b], sin_[b]).wait()

        def out_start(c, b):
            start = chunk_start(c)
            pltpu.async_copy(rout[b], out_hbm.at[:, pl.ds(start, _C)],
                             sout[b])

        def out_wait(c, b):
            start = chunk_start(c)
            pltpu.make_async_copy(
                rout[b], out_hbm.at[:, pl.ds(start, _C)], sout[b]).wait()

        def compute(b):
            ray_i = rin[b]
            ray_o = rout[b]
            idx_v = idxv[b]
            mask_v = mskv[b]

            def group(g, c2):
                loaded = []
                for u in range(2):
                    sl = pl.ds((g * 2 + u) * _L, _L)
                    ix = idx_v[sl] & idx_mask
                    p01 = plsc.load_gather(t01_v, [ix])
                    p2t0 = plsc.load_gather(t2t0_v, [ix])
                    pt12 = plsc.load_gather(t12_v, [ix])
                    psc = plsc.load_gather(tsc_v, [ix])
                    mf = mask_v[sl]
                    o = [ray_i[c, sl] for c in range(3)]
                    d = [ray_i[c + 3, sl] for c in range(3)]
                    e = [ray_i[c + 6, sl] for c in range(2)]
                    loaded.append((sl, p01, p2t0, pt12, psc, mf, o, d, e))
                results = []
                for sl, p01, p2t0, pt12, psc, mf, o, d, e in loaded:
                    w0 = hi(p01)
                    w1 = lo(p01)
                    w2 = hi(p2t0)
                    t0 = lo(p2t0)
                    t1 = hi(pt12)
                    t2 = lo(pt12)
                    sm = hi(psc) * mf
                    c1m = lo(psc) * mf
                    dx, dy, dz = d
                    cx = w1 * dz - w2 * dy
                    cy = w2 * dx - w0 * dz
                    cz = w0 * dy - w1 * dx
                    ex = w1 * cz - w2 * cy
                    ey = w2 * cx - w0 * cz
                    ez = w0 * cy - w1 * cx
                    results.append((
                        sl,
                        [o[0] + t0 * mf, o[1] + t1 * mf, o[2] + t2 * mf,
                         dx + sm * cx + c1m * ex,
                         dy + sm * cy + c1m * ey,
                         dz + sm * cz + c1m * ez] + e))
                for sl, vals in results:
                    for c in range(8):
                        ray_o[c, sl] = vals[c]
                return c2

            lax.fori_loop(0, _C // (2 * _L), group, 0)

        in_start(0, 0)
        in_start(1, 1)
        in_wait(0, 0)
        compute(0)
        out_start(0, 0)
        in_start(2, 0)
        in_wait(1, 1)
        compute(1)
        out_start(1, 1)
        in_start(3, 1)

        def pair(j, carry):
            a = 2 * j
            in_wait(a, 0)
            out_wait(a - 2, 0)
            compute(0)
            out_start(a, 0)
            in_start(a + 2, 0)
            in_wait(a + 1, 1)
            out_wait(a - 1, 1)
            compute(1)
            out_start(a + 1, 1)
            in_start(a + 3, 1)
            return carry

        J = (K - 1) // 2
        lax.fori_loop(1, J + 1, pair, 0)
        in_wait(2 * J + 2, 0)
        in_wait(2 * J + 3, 1)
        out_wait(2 * J, 0)
        out_wait(2 * J + 1, 1)

    return sc_apply


def kernel(w, v, theta, rays, image_indices, depth_mask):
    B = rays.shape[0]
    n = theta.shape[0]
    t01, t2t0, t12, tsc = _build_tables(
        w.astype(jnp.float32), v.astype(jnp.float32),
        theta.astype(jnp.float32))
    raysT = rays.T
    idx = image_indices.reshape(-1).astype(jnp.int32)
    maskf = (depth_mask.reshape(-1) == 1).astype(jnp.float32)
    sc_apply = _make_sc_apply(B, n)
    outT = sc_apply(t01, t2t0, t12, tsc, raysT, idx, maskf)
    return outT.T


# speedup vs baseline: 3.0795x; 1.0302x over previous
import functools

import jax
import jax.numpy as jnp
from jax import lax
from jax.experimental import pallas as pl
from jax.experimental.pallas import tpu as pltpu, tpu_sc as plsc

_NC = 2
_NS = 16
_L = 16
_C = 1536
_U = 3


def _pack2(a, b):
    ua = lax.bitcast_convert_type(
        a.astype(jnp.bfloat16).astype(jnp.float32), jnp.uint32)
    ub = lax.bitcast_convert_type(
        b.astype(jnp.bfloat16).astype(jnp.float32), jnp.uint32)
    return lax.bitcast_convert_type(ua | (ub >> 16), jnp.int32)


def _table_body(w_ref, v_ref, th_ref, o_w01, o_w2t0, o_t12, o_sc):
    w0, w1, w2 = w_ref[0], w_ref[1], w_ref[2]
    v0, v1, v2 = v_ref[0], v_ref[1], v_ref[2]
    th = th_ref[...]
    s = jnp.sin(th)
    c1 = 1.0 - jnp.cos(th)
    tms = th - s
    a0 = w1 * v2 - w2 * v1
    a1 = w2 * v0 - w0 * v2
    a2 = w0 * v1 - w1 * v0
    b0 = w1 * a2 - w2 * a1
    b1 = w2 * a0 - w0 * a2
    b2 = w0 * a1 - w1 * a0
    t0 = th * v0 + c1 * a0 + tms * b0
    t1 = th * v1 + c1 * a1 + tms * b1
    t2 = th * v2 + c1 * a2 + tms * b2
    o_w01[...] = _pack2(w0, w1)
    o_w2t0[...] = _pack2(w2, t0)
    o_t12[...] = _pack2(t1, t2)
    o_sc[...] = _pack2(s, c1)


def _build_tables(w, v, theta):
    n = theta.shape[0]
    r = n // 128
    wt = w.T.reshape(3, r, 128)
    vt = v.T.reshape(3, r, 128)
    th = theta.reshape(r, 128)
    o = jax.ShapeDtypeStruct((r, 128), jnp.int32)
    t01, t2t0, t12, tsc = pl.pallas_call(
        _table_body, out_shape=[o, o, o, o])(wt, vt, th)
    return t01.reshape(n), t2t0.reshape(n), t12.reshape(n), tsc.reshape(n)


def _make_sc_apply(B, n):
    W = _NC * _NS
    K = -(-B // (_C * W))
    last = (B // 128) * 128 - _C + 128
    idx_mask = n - 1

    mesh = plsc.VectorSubcoreMesh(
        core_axis_name="c", subcore_axis_name="s",
        num_cores=_NC, num_subcores=_NS)

    @functools.partial(
        pl.kernel,
        out_type=jax.ShapeDtypeStruct((8, B), jnp.float32),
        mesh=mesh,
        scratch_types=[
            pltpu.VMEM((n,), jnp.int32),
            pltpu.VMEM((n,), jnp.int32),
            pltpu.VMEM((n,), jnp.int32),
            pltpu.VMEM((n,), jnp.int32),
            [pltpu.VMEM((8, _C), jnp.float32) for _ in range(2)],
            [pltpu.VMEM((8, _C), jnp.float32) for _ in range(2)],
            [pltpu.VMEM((_C,), jnp.int32) for _ in range(2)],
            [pltpu.VMEM((_C,), jnp.float32) for _ in range(2)],
            [pltpu.SemaphoreType.DMA for _ in range(2)],
            [pltpu.SemaphoreType.DMA for _ in range(2)],
        ],
        compiler_params=pltpu.CompilerParams(
            needs_layout_passes=False, use_tc_tiling_on_sc=True),
    )
    def sc_apply(t01_hbm, t2t0_hbm, t12_hbm, tsc_hbm, raysT_hbm, idx_hbm,
                 mask_hbm, out_hbm, t01_v, t2t0_v, t12_v, tsc_v,
                 rin, rout, idxv, mskv, sin_, sout):
        wid = lax.axis_index("s") * _NC + lax.axis_index("c")
        pltpu.sync_copy(t01_hbm, t01_v)
        pltpu.sync_copy(t2t0_hbm, t2t0_v)
        pltpu.sync_copy(t12_hbm, t12_v)
        pltpu.sync_copy(tsc_hbm, tsc_v)

        def hi(p):
            return plsc.bitcast(p & jnp.int32(-65536), jnp.float32)

        def lo(p):
            return plsc.bitcast(p << 16, jnp.float32)

        def chunk_start(c):
            return jnp.minimum((wid * K + jnp.minimum(c, K - 1)) * _C, last)

        def in_start(c, b):
            start = chunk_start(c)
            pltpu.async_copy(idx_hbm.at[pl.ds(start, _C)], idxv[b], sin_[b])
            pltpu.async_copy(mask_hbm.at[pl.ds(start, _C)], mskv[b], sin_[b])
            pltpu.async_copy(raysT_hbm.at[:, pl.ds(start, _C)], rin[b],
                             sin_[b])

        def in_wait(c, b):
            start = chunk_start(c)
            pltpu.make_async_copy(
                idx_hbm.at[pl.ds(start, _C)], idxv[b], sin_[b]).wait()
            pltpu.make_async_copy(
                mask_hbm.at[pl.ds(start, _C)], mskv[b], sin_[b]).wait()
            pltpu.make_async_copy(
                raysT_hbm.at[:, pl.ds(start, _C)], rin[b], sin_[b]).wait()

        def out_start(c, b):
            start = chunk_start(c)
            pltpu.async_copy(rout[b], out_hbm.at[:, pl.ds(start, _C)],
                             sout[b])

        def out_wait(c, b):
            start = chunk_start(c)
            pltpu.make_async_copy(
                rout[b], out_hbm.at[:, pl.ds(start, _C)], sout[b]).wait()

        def compute(b):
            ray_i = rin[b]
            ray_o = rout[b]
            idx_v = idxv[b]
            mask_v = mskv[b]

            def group(g, c2):
                loaded = []
                for u in range(_U):
                    sl = pl.ds((g * _U + u) * _L, _L)
                    ix = idx_v[sl] & idx_mask
                    p01 = plsc.load_gather(t01_v, [ix])
                    p2t0 = plsc.load_gather(t2t0_v, [ix])
                    pt12 = plsc.load_gather(t12_v, [ix])
                    psc = plsc.load_gather(tsc_v, [ix])
                    mf = mask_v[sl]
                    o = [ray_i[c, sl] for c in range(3)]
                    d = [ray_i[c + 3, sl] for c in range(3)]
                    e = [ray_i[c + 6, sl] for c in range(2)]
                    loaded.append((sl, p01, p2t0, pt12, psc, mf, o, d, e))
                results = []
                for sl, p01, p2t0, pt12, psc, mf, o, d, e in loaded:
                    w0 = hi(p01)
                    w1 = lo(p01)
                    w2 = hi(p2t0)
                    t0 = lo(p2t0)
                    t1 = hi(pt12)
                    t2 = lo(pt12)
                    sm = hi(psc) * mf
                    c1m = lo(psc) * mf
                    dx, dy, dz = d
                    cx = w1 * dz - w2 * dy
                    cy = w2 * dx - w0 * dz
                    cz = w0 * dy - w1 * dx
                    ex = w1 * cz - w2 * cy
                    ey = w2 * cx - w0 * cz
                    ez = w0 * cy - w1 * cx
                    results.append((
                        sl,
                        [o[0] + t0 * mf, o[1] + t1 * mf, o[2] + t2 * mf,
                         dx + sm * cx + c1m * ex,
                         dy + sm * cy + c1m * ey,
                         dz + sm * cz + c1m * ez] + e))
                for sl, vals in results:
                    for c in range(8):
                        ray_o[c, sl] = vals[c]
                return c2

            lax.fori_loop(0, _C // (_U * _L), group, 0)

        in_start(0, 0)
        in_start(1, 1)
        in_wait(0, 0)
        compute(0)
        out_start(0, 0)
        in_start(2, 0)
        in_wait(1, 1)
        compute(1)
        out_start(1, 1)
        in_start(3, 1)

        def pair(j, carry):
            a = 2 * j
            in_wait(a, 0)
            out_wait(a - 2, 0)
            compute(0)
            out_start(a, 0)
            in_start(a + 2, 0)
            in_wait(a + 1, 1)
            out_wait(a - 1, 1)
            compute(1)
            out_start(a + 1, 1)
            in_start(a + 3, 1)
            return carry

        J = (K - 1) // 2
        lax.fori_loop(1, J + 1, pair, 0)
        in_wait(2 * J + 2, 0)
        in_wait(2 * J + 3, 1)
        out_wait(2 * J, 0)
        out_wait(2 * J + 1, 1)

    return sc_apply


def kernel(w, v, theta, rays, image_indices, depth_mask):
    B = rays.shape[0]
    n = theta.shape[0]
    t01, t2t0, t12, tsc = _build_tables(
        w.astype(jnp.float32), v.astype(jnp.float32),
        theta.astype(jnp.float32))
    raysT = rays.T
    idx = image_indices.reshape(-1).astype(jnp.int32)
    maskf = (depth_mask.reshape(-1) == 1).astype(jnp.float32)
    sc_apply = _make_sc_apply(B, n)
    outT = sc_apply(t01, t2t0, t12, tsc, raysT, idx, maskf)
    return outT.T


# speedup vs baseline: 3.0852x; 1.0019x over previous
import functools

import jax
import jax.numpy as jnp
from jax import lax
from jax.experimental import pallas as pl
from jax.experimental.pallas import tpu as pltpu, tpu_sc as plsc

_NC = 2
_NS = 16
_L = 16
_C = 1536
_U = 4


def _pack2(a, b):
    ua = lax.bitcast_convert_type(
        a.astype(jnp.bfloat16).astype(jnp.float32), jnp.uint32)
    ub = lax.bitcast_convert_type(
        b.astype(jnp.bfloat16).astype(jnp.float32), jnp.uint32)
    return lax.bitcast_convert_type(ua | (ub >> 16), jnp.int32)


def _table_body(w_ref, v_ref, th_ref, o_w01, o_w2t0, o_t12, o_sc):
    w0, w1, w2 = w_ref[0], w_ref[1], w_ref[2]
    v0, v1, v2 = v_ref[0], v_ref[1], v_ref[2]
    th = th_ref[...]
    s = jnp.sin(th)
    c1 = 1.0 - jnp.cos(th)
    tms = th - s
    a0 = w1 * v2 - w2 * v1
    a1 = w2 * v0 - w0 * v2
    a2 = w0 * v1 - w1 * v0
    b0 = w1 * a2 - w2 * a1
    b1 = w2 * a0 - w0 * a2
    b2 = w0 * a1 - w1 * a0
    t0 = th * v0 + c1 * a0 + tms * b0
    t1 = th * v1 + c1 * a1 + tms * b1
    t2 = th * v2 + c1 * a2 + tms * b2
    o_w01[...] = _pack2(w0, w1)
    o_w2t0[...] = _pack2(w2, t0)
    o_t12[...] = _pack2(t1, t2)
    o_sc[...] = _pack2(s, c1)


def _build_tables(w, v, theta):
    n = theta.shape[0]
    r = n // 128
    wt = w.T.reshape(3, r, 128)
    vt = v.T.reshape(3, r, 128)
    th = theta.reshape(r, 128)
    o = jax.ShapeDtypeStruct((r, 128), jnp.int32)
    t01, t2t0, t12, tsc = pl.pallas_call(
        _table_body, out_shape=[o, o, o, o])(wt, vt, th)
    return t01.reshape(n), t2t0.reshape(n), t12.reshape(n), tsc.reshape(n)


def _make_sc_apply(B, n):
    W = _NC * _NS
    K = -(-B // (_C * W))
    last = (B // 128) * 128 - _C + 128
    idx_mask = n - 1

    mesh = plsc.VectorSubcoreMesh(
        core_axis_name="c", subcore_axis_name="s",
        num_cores=_NC, num_subcores=_NS)

    @functools.partial(
        pl.kernel,
        out_type=jax.ShapeDtypeStruct((8, B), jnp.float32),
        mesh=mesh,
        scratch_types=[
            pltpu.VMEM((n,), jnp.int32),
            pltpu.VMEM((n,), jnp.int32),
            pltpu.VMEM((n,), jnp.int32),
            pltpu.VMEM((n,), jnp.int32),
            [pltpu.VMEM((8, _C), jnp.float32) for _ in range(2)],
            [pltpu.VMEM((8, _C), jnp.float32) for _ in range(2)],
            [pltpu.VMEM((_C,), jnp.int32) for _ in range(2)],
            [pltpu.VMEM((_C,), jnp.float32) for _ in range(2)],
            [pltpu.SemaphoreType.DMA for _ in range(2)],
            [pltpu.SemaphoreType.DMA for _ in range(2)],
        ],
        compiler_params=pltpu.CompilerParams(
            needs_layout_passes=False, use_tc_tiling_on_sc=True),
    )
    def sc_apply(t01_hbm, t2t0_hbm, t12_hbm, tsc_hbm, raysT_hbm, idx_hbm,
                 mask_hbm, out_hbm, t01_v, t2t0_v, t12_v, tsc_v,
                 rin, rout, idxv, mskv, sin_, sout):
        wid = lax.axis_index("s") * _NC + lax.axis_index("c")
        pltpu.sync_copy(t01_hbm, t01_v)
        pltpu.sync_copy(t2t0_hbm, t2t0_v)
        pltpu.sync_copy(t12_hbm, t12_v)
        pltpu.sync_copy(tsc_hbm, tsc_v)

        def hi(p):
            return plsc.bitcast(p & jnp.int32(-65536), jnp.float32)

        def lo(p):
            return plsc.bitcast(p << 16, jnp.float32)

        def chunk_start(c):
            return jnp.minimum((wid * K + jnp.minimum(c, K - 1)) * _C, last)

        def in_start(c, b):
            start = chunk_start(c)
            pltpu.async_copy(idx_hbm.at[pl.ds(start, _C)], idxv[b], sin_[b])
            pltpu.async_copy(mask_hbm.at[pl.ds(start, _C)], mskv[b], sin_[b])
            pltpu.async_copy(raysT_hbm.at[:, pl.ds(start, _C)], rin[b],
                             sin_[b])

        def in_wait(c, b):
            start = chunk_start(c)
            pltpu.make_async_copy(
                idx_hbm.at[pl.ds(start, _C)], idxv[b], sin_[b]).wait()
            pltpu.make_async_copy(
                mask_hbm.at[pl.ds(start, _C)], mskv[b], sin_[b]).wait()
            pltpu.make_async_copy(
                raysT_hbm.at[:, pl.ds(start, _C)], rin[b], sin_[b]).wait()

        def out_start(c, b):
            start = chunk_start(c)
            pltpu.async_copy(rout[b], out_hbm.at[:, pl.ds(start, _C)],
                             sout[b])

        def out_wait(c, b):
            start = chunk_start(c)
            pltpu.make_async_copy(
                rout[b], out_hbm.at[:, pl.ds(start, _C)], sout[b]).wait()

        def compute(b):
            ray_i = rin[b]
            ray_o = rout[b]
            idx_v = idxv[b]
            mask_v = mskv[b]

            def group(g, c2):
                loaded = []
                for u in range(_U):
                    sl = pl.ds((g * _U + u) * _L, _L)
                    ix = idx_v[sl] & idx_mask
                    p01 = plsc.load_gather(t01_v, [ix])
                    p2t0 = plsc.load_gather(t2t0_v, [ix])
                    pt12 = plsc.load_gather(t12_v, [ix])
                    psc = plsc.load_gather(tsc_v, [ix])
                    mf = mask_v[sl]
                    o = [ray_i[c, sl] for c in range(3)]
                    d = [ray_i[c + 3, sl] for c in range(3)]
                    e = [ray_i[c + 6, sl] for c in range(2)]
                    loaded.append((sl, p01, p2t0, pt12, psc, mf, o, d, e))
                results = []
                for sl, p01, p2t0, pt12, psc, mf, o, d, e in loaded:
                    w0 = hi(p01)
                    w1 = lo(p01)
                    w2 = hi(p2t0)
                    t0 = lo(p2t0)
                    t1 = hi(pt12)
                    t2 = lo(pt12)
                    sm = hi(psc) * mf
                    c1m = lo(psc) * mf
                    dx, dy, dz = d
                    cx = w1 * dz - w2 * dy
                    cy = w2 * dx - w0 * dz
                    cz = w0 * dy - w1 * dx
                    ex = w1 * cz - w2 * cy
                    ey = w2 * cx - w0 * cz
                    ez = w0 * cy - w1 * cx
                    results.append((
                        sl,
                        [o[0] + t0 * mf, o[1] + t1 * mf, o[2] + t2 * mf,
                         dx + sm * cx + c1m * ex,
                         dy + sm * cy + c1m * ey,
                         dz + sm * cz + c1m * ez] + e))
                for sl, vals in results:
                    for c in range(8):
                        ray_o[c, sl] = vals[c]
                return c2

            lax.fori_loop(0, _C // (_U * _L), group, 0)

        in_start(0, 0)
        in_start(1, 1)
        in_wait(0, 0)
        compute(0)
        out_start(0, 0)
        in_start(2, 0)
        in_wait(1, 1)
        compute(1)
        out_start(1, 1)
        in_start(3, 1)

        def pair(j, carry):
            a = 2 * j
            in_wait(a, 0)
            out_wait(a - 2, 0)
            compute(0)
            out_start(a, 0)
            in_start(a + 2, 0)
            in_wait(a + 1, 1)
            out_wait(a - 1, 1)
            compute(1)
            out_start(a + 1, 1)
            in_start(a + 3, 1)
            return carry

        J = (K - 1) // 2
        lax.fori_loop(1, J + 1, pair, 0)
        in_wait(2 * J + 2, 0)
        in_wait(2 * J + 3, 1)
        out_wait(2 * J, 0)
        out_wait(2 * J + 1, 1)

    return sc_apply


def kernel(w, v, theta, rays, image_indices, depth_mask):
    B = rays.shape[0]
    n = theta.shape[0]
    t01, t2t0, t12, tsc = _build_tables(
        w.astype(jnp.float32), v.astype(jnp.float32),
        theta.astype(jnp.float32))
    raysT = rays.T
    idx = image_indices.reshape(-1).astype(jnp.int32)
    maskf = (depth_mask.reshape(-1) == 1).astype(jnp.float32)
    sc_apply = _make_sc_apply(B, n)
    outT = sc_apply(t01, t2t0, t12, tsc, raysT, idx, maskf)
    return outT.T
